# Initial kernel scaffold; baseline (speedup 1.0000x reference)
import jax
import jax.numpy as jnp
from jax.experimental import pallas as pl


def kernel(x, edge_index, edge_weight, W, b):
    raise NotImplementedError("write your pallas kernel here")


# speedup vs baseline: 11.0553x; 11.0553x over previous
import functools

import jax
import jax.numpy as jnp
from jax import lax
from jax.experimental import pallas as pl
from jax.experimental.pallas import tpu as pltpu
from jax.experimental.pallas import tpu_sc as plsc

N = 10000
D = 128
DH = D // 2
NPAD = 10240
NC, NS, L = 2, 16, 16
NW = NC * NS
CHUNK = 128


def _ceil_to(x, m):
    return (x + m - 1) // m * m


def _deg_body(nchunk, dst_hbm, ew_hbm, out_hbm, dst_v, ew_v, zero_v, deg_sp):
    cid = lax.axis_index("c")
    sid = lax.axis_index("s")
    wid = cid * NS + sid
    pltpu.sync_copy(dst_hbm.at[wid], dst_v)
    pltpu.sync_copy(ew_hbm.at[wid], ew_v)

    nz = NPAD // NS

    def _z(i, _):
        zero_v[pl.ds(i * L, L)] = jnp.zeros((L,), jnp.float32)
        return 0

    lax.fori_loop(0, nz // L, _z, 0)
    pltpu.sync_copy(zero_v, deg_sp.at[pl.ds(sid * nz, nz)])
    plsc.subcore_barrier()

    def _chunk(k, _):
        pltpu.sync_copy(ew_v.at[k], deg_sp.at[dst_v.at[k]], add=True)
        return 0

    lax.fori_loop(0, nchunk, _chunk, 0)
    plsc.subcore_barrier()

    @pl.when(sid == 0)
    def _():
        pltpu.sync_copy(deg_sp, out_hbm.at[cid])


def _prop_body(nchunk, src_hbm, dst_hbm, ew_hbm, dis_hbm, h_hbm, b_hbm,
               out_hbm, src_v, dst_v, ew_v, dis_v, idx_v, norm_v, rows_v,
               brow_v, b_v, acc_sp, sem):
    cid = lax.axis_index("c")
    sid = lax.axis_index("s")
    pltpu.sync_copy(src_hbm.at[sid], src_v)
    pltpu.sync_copy(dst_hbm.at[sid], dst_v)
    pltpu.sync_copy(ew_hbm.at[sid], ew_v)
    pltpu.sync_copy(dis_hbm, dis_v)
    pltpu.sync_copy(b_hbm.at[cid], b_v)

    bvecs = [b_v[pl.ds(q * L, L)] for q in range(DH // L)]

    def _br(i, _):
        for q in range(DH // L):
            brow_v[i, pl.ds(q * L, L)] = bvecs[q]
        return 0

    lax.fori_loop(0, CHUNK, _br, 0)
    for dblk in range(NPAD // NS // CHUNK):
        pltpu.sync_copy(brow_v,
                        acc_sp.at[pl.ds(sid * (NPAD // NS) + dblk * CHUNK, CHUNK)])
    plsc.subcore_barrier()

    table_off = cid * N

    def _chunk(k, _):
        for g in range(CHUNK // L):
            s16 = src_v[k, pl.ds(g * L, L)]
            d16 = dst_v[k, pl.ds(g * L, L)]
            e16 = ew_v[k, pl.ds(g * L, L)]
            n16 = plsc.load_gather(dis_v, [s16]) * e16 * plsc.load_gather(dis_v, [d16])
            norm_v[pl.ds(g * L, L)] = n16
            idx_v[pl.ds(g * L, L)] = s16 + table_off
        pltpu.async_copy(h_hbm.at[idx_v], rows_v, sem).wait()

        def _scale(g, _):
            n16 = norm_v[pl.ds(g * L, L)]
            for lane in range(L):
                s = n16[lane]
                jj = g * L + lane
                for q in range(DH // L):
                    rows_v[jj, pl.ds(q * L, L)] = rows_v[jj, pl.ds(q * L, L)] * s
            return 0

        lax.fori_loop(0, CHUNK // L, _scale, 0)
        pltpu.sync_copy(rows_v, acc_sp.at[dst_v.at[k]], add=True)
        return 0

    lax.fori_loop(0, nchunk, _chunk, 0)
    plsc.subcore_barrier()
    rows_per_tile = NPAD // NS
    pltpu.sync_copy(acc_sp.at[pl.ds(sid * rows_per_tile, rows_per_tile)],
                    out_hbm.at[cid, pl.ds(sid * rows_per_tile, rows_per_tile)])


def _mm_body(parts_ref, x_ref, w_ref, dis_ref, h_ref):
    deg = jnp.sum(parts_ref[...], axis=0, keepdims=True)
    dis = jnp.where(deg > 0.0, lax.rsqrt(jnp.maximum(deg, 1e-12)), 0.0)
    dis_ref[...] = dis
    h = jnp.dot(x_ref[...], w_ref[...], preferred_element_type=jnp.float32)
    h_ref[0] = h[:, :DH]
    h_ref[1] = h[:, DH:]


def kernel(x, edge_index, edge_weight, W, b):
    E = edge_index.shape[1]
    src = edge_index[0].astype(jnp.int32)
    dst = edge_index[1].astype(jnp.int32)
    loop_idx = jnp.arange(N, dtype=jnp.int32)
    e_tot = E + N
    e_pad = _ceil_to(e_tot, NS * CHUNK)
    npad_e = e_pad - e_tot
    src_all = jnp.concatenate([src, loop_idx,
                               jnp.zeros((npad_e,), jnp.int32)])
    dst_all = jnp.concatenate([dst, loop_idx,
                               jnp.zeros((npad_e,), jnp.int32)])
    ew_all = jnp.concatenate([edge_weight.astype(jnp.float32),
                              jnp.ones((N,), jnp.float32),
                              jnp.zeros((npad_e,), jnp.float32)])
    nchunk32 = e_pad // (NW * CHUNK)
    nchunk16 = e_pad // (NS * CHUNK)
    src_r16 = src_all.reshape(NS, nchunk16, CHUNK)
    dst_r16 = dst_all.reshape(NS, nchunk16, CHUNK)
    ew_r16 = ew_all.reshape(NS, nchunk16, CHUNK)
    dst_r32 = dst_all.reshape(NW, nchunk32, CHUNK)
    ew_r32 = ew_all.reshape(NW, nchunk32, CHUNK)

    mesh = plsc.VectorSubcoreMesh(core_axis_name="c", subcore_axis_name="s")

    deg_parts = pl.kernel(
        functools.partial(_deg_body, nchunk32),
        out_type=jax.ShapeDtypeStruct((NC, NPAD), jnp.float32),
        mesh=mesh,
        scratch_types=[
            pltpu.VMEM((nchunk32, CHUNK), jnp.int32),
            pltpu.VMEM((nchunk32, CHUNK), jnp.float32),
            pltpu.VMEM((NPAD // NS,), jnp.float32),
            pltpu.VMEM_SHARED((NPAD,), jnp.float32),
        ],
    )(dst_r32, ew_r32)

    dis2d, h_half = pl.pallas_call(
        _mm_body,
        out_shape=[
            jax.ShapeDtypeStruct((1, NPAD), jnp.float32),
            jax.ShapeDtypeStruct((NC, N, DH), jnp.float32),
        ],
    )(deg_parts, x, W)
    dis = dis2d.reshape(NPAD)
    h_tab = h_half.reshape(NC * N, DH)
    b_r = b.astype(jnp.float32).reshape(NC, DH)

    acc_parts = pl.kernel(
        functools.partial(_prop_body, nchunk16),
        out_type=jax.ShapeDtypeStruct((NC, NPAD, DH), jnp.float32),
        mesh=mesh,
        compiler_params=pltpu.CompilerParams(needs_layout_passes=False,
                                             use_tc_tiling_on_sc=False),
        scratch_types=[
            pltpu.VMEM((nchunk16, CHUNK), jnp.int32),
            pltpu.VMEM((nchunk16, CHUNK), jnp.int32),
            pltpu.VMEM((nchunk16, CHUNK), jnp.float32),
            pltpu.VMEM((NPAD,), jnp.float32),
            pltpu.VMEM((CHUNK,), jnp.int32),
            pltpu.VMEM((CHUNK,), jnp.float32),
            pltpu.VMEM((CHUNK, DH), jnp.float32),
            pltpu.VMEM((CHUNK, DH), jnp.float32),
            pltpu.VMEM((DH,), jnp.float32),
            pltpu.VMEM_SHARED((NPAD, DH), jnp.float32),
            pltpu.SemaphoreType.DMA,
        ],
    )(src_r16, dst_r16, ew_r16, dis, h_tab, b_r)

    out = jnp.concatenate([acc_parts[0, :N], acc_parts[1, :N]], axis=1)
    return out


# speedup vs baseline: 18.5054x; 1.6739x over previous
import functools

import jax
import jax.numpy as jnp
from jax import lax
from jax.experimental import pallas as pl
from jax.experimental.pallas import tpu as pltpu
from jax.experimental.pallas import tpu_sc as plsc

N = 10000
D = 128
DH = D // 2
NPAD = 10240
NC, NS, L = 2, 16, 16
NW = NC * NS
CHUNK = 128


def _ceil_to(x, m):
    return (x + m - 1) // m * m


def _deg_body(nchunk, dst_hbm, ew_hbm, out_hbm, dst_v, ew_v, zero_v, deg_sp):
    cid = lax.axis_index("c")
    sid = lax.axis_index("s")
    wid = cid * NS + sid
    pltpu.sync_copy(dst_hbm.at[wid], dst_v)
    pltpu.sync_copy(ew_hbm.at[wid], ew_v)

    nz = NPAD // NS

    def _z(i, _):
        zero_v[pl.ds(i * L, L)] = jnp.zeros((L,), jnp.float32)
        return 0

    lax.fori_loop(0, nz // L, _z, 0)
    pltpu.sync_copy(zero_v, deg_sp.at[pl.ds(sid * nz, nz)])
    plsc.subcore_barrier()

    def _chunk(k, _):
        pltpu.sync_copy(ew_v.at[k], deg_sp.at[dst_v.at[k]], add=True)
        return 0

    lax.fori_loop(0, nchunk, _chunk, 0)
    plsc.subcore_barrier()

    @pl.when(sid == 0)
    def _():
        pltpu.sync_copy(deg_sp, out_hbm.at[cid])


def _prop_body(nchunk, src_hbm, dst_hbm, ew_hbm, dis_hbm, h_hbm, b_hbm,
               out_hbm, src_v, dst_v, ew_v, dis_v, idx_v, norm_v, rows_v,
               brow_v, b_v, acc_sp, sem):
    cid = lax.axis_index("c")
    sid = lax.axis_index("s")
    pltpu.sync_copy(src_hbm.at[sid], src_v)
    pltpu.sync_copy(dst_hbm.at[sid], dst_v)
    pltpu.sync_copy(ew_hbm.at[sid], ew_v)
    pltpu.sync_copy(dis_hbm, dis_v)
    pltpu.sync_copy(b_hbm.at[cid], b_v)

    bvecs = [b_v[pl.ds(q * L, L)] for q in range(DH // L)]

    def _br(i, _):
        for q in range(DH // L):
            brow_v[i, pl.ds(q * L, L)] = bvecs[q]
        return 0

    lax.fori_loop(0, CHUNK, _br, 0)
    for dblk in range(NPAD // NS // CHUNK):
        pltpu.sync_copy(brow_v,
                        acc_sp.at[pl.ds(sid * (NPAD // NS) + dblk * CHUNK, CHUNK)])
    plsc.subcore_barrier()

    table_off = cid * N

    def _chunk(k, _):
        for g in range(CHUNK // L):
            s16 = src_v[k, pl.ds(g * L, L)]
            d16 = dst_v[k, pl.ds(g * L, L)]
            e16 = ew_v[k, pl.ds(g * L, L)]
            n16 = plsc.load_gather(dis_v, [s16]) * e16 * plsc.load_gather(dis_v, [d16])
            norm_v[pl.ds(g * L, L)] = n16
            idx_v[pl.ds(g * L, L)] = s16 + table_off
        pltpu.async_copy(h_hbm.at[idx_v], rows_v, sem).wait()

        for g in range(CHUNK // L):
            n16 = norm_v[pl.ds(g * L, L)]
            for lane in range(L):
                s = n16[lane]
                jj = g * L + lane
                for q in range(DH // L):
                    rows_v[jj, pl.ds(q * L, L)] = rows_v[jj, pl.ds(q * L, L)] * s
        pltpu.sync_copy(rows_v, acc_sp.at[dst_v.at[k]], add=True)
        return 0

    lax.fori_loop(0, nchunk, _chunk, 0)
    plsc.subcore_barrier()
    rows_per_tile = NPAD // NS
    pltpu.sync_copy(acc_sp.at[pl.ds(sid * rows_per_tile, rows_per_tile)],
                    out_hbm.at[cid, pl.ds(sid * rows_per_tile, rows_per_tile)])


def _mm_body(parts_ref, x_ref, w_ref, dis_ref, h_ref):
    deg = jnp.sum(parts_ref[...], axis=0, keepdims=True)
    dis = jnp.where(deg > 0.0, lax.rsqrt(jnp.maximum(deg, 1e-12)), 0.0)
    dis_ref[...] = dis
    h = jnp.dot(x_ref[...], w_ref[...], preferred_element_type=jnp.float32)
    h_ref[0] = h[:, :DH]
    h_ref[1] = h[:, DH:]


def kernel(x, edge_index, edge_weight, W, b):
    E = edge_index.shape[1]
    src = edge_index[0].astype(jnp.int32)
    dst = edge_index[1].astype(jnp.int32)
    loop_idx = jnp.arange(N, dtype=jnp.int32)
    e_tot = E + N
    e_pad = _ceil_to(e_tot, NS * CHUNK)
    npad_e = e_pad - e_tot
    src_all = jnp.concatenate([src, loop_idx,
                               jnp.zeros((npad_e,), jnp.int32)])
    dst_all = jnp.concatenate([dst, loop_idx,
                               jnp.zeros((npad_e,), jnp.int32)])
    ew_all = jnp.concatenate([edge_weight.astype(jnp.float32),
                              jnp.ones((N,), jnp.float32),
                              jnp.zeros((npad_e,), jnp.float32)])
    nchunk32 = e_pad // (NW * CHUNK)
    nchunk16 = e_pad // (NS * CHUNK)
    src_r16 = src_all.reshape(NS, nchunk16, CHUNK)
    dst_r16 = dst_all.reshape(NS, nchunk16, CHUNK)
    ew_r16 = ew_all.reshape(NS, nchunk16, CHUNK)
    dst_r32 = dst_all.reshape(NW, nchunk32, CHUNK)
    ew_r32 = ew_all.reshape(NW, nchunk32, CHUNK)

    mesh = plsc.VectorSubcoreMesh(core_axis_name="c", subcore_axis_name="s")

    deg_parts = pl.kernel(
        functools.partial(_deg_body, nchunk32),
        out_type=jax.ShapeDtypeStruct((NC, NPAD), jnp.float32),
        mesh=mesh,
        scratch_types=[
            pltpu.VMEM((nchunk32, CHUNK), jnp.int32),
            pltpu.VMEM((nchunk32, CHUNK), jnp.float32),
            pltpu.VMEM((NPAD // NS,), jnp.float32),
            pltpu.VMEM_SHARED((NPAD,), jnp.float32),
        ],
    )(dst_r32, ew_r32)

    dis2d, h_half = pl.pallas_call(
        _mm_body,
        out_shape=[
            jax.ShapeDtypeStruct((1, NPAD), jnp.float32),
            jax.ShapeDtypeStruct((NC, N, DH), jnp.float32),
        ],
    )(deg_parts, x, W)
    dis = dis2d.reshape(NPAD)
    h_tab = h_half.reshape(NC * N, DH)
    b_r = b.astype(jnp.float32).reshape(NC, DH)

    acc_parts = pl.kernel(
        functools.partial(_prop_body, nchunk16),
        out_type=jax.ShapeDtypeStruct((NC, NPAD, DH), jnp.float32),
        mesh=mesh,
        compiler_params=pltpu.CompilerParams(needs_layout_passes=False,
                                             use_tc_tiling_on_sc=False),
        scratch_types=[
            pltpu.VMEM((nchunk16, CHUNK), jnp.int32),
            pltpu.VMEM((nchunk16, CHUNK), jnp.int32),
            pltpu.VMEM((nchunk16, CHUNK), jnp.float32),
            pltpu.VMEM((NPAD,), jnp.float32),
            pltpu.VMEM((CHUNK,), jnp.int32),
            pltpu.VMEM((CHUNK,), jnp.float32),
            pltpu.VMEM((CHUNK, DH), jnp.float32),
            pltpu.VMEM((CHUNK, DH), jnp.float32),
            pltpu.VMEM((DH,), jnp.float32),
            pltpu.VMEM_SHARED((NPAD, DH), jnp.float32),
            pltpu.SemaphoreType.DMA,
        ],
    )(src_r16, dst_r16, ew_r16, dis, h_tab, b_r)

    out = jnp.concatenate([acc_parts[0, :N], acc_parts[1, :N]], axis=1)
    return out


# speedup vs baseline: 18.9172x; 1.0223x over previous
import functools

import jax
import jax.numpy as jnp
from jax import lax
from jax.experimental import pallas as pl
from jax.experimental.pallas import tpu as pltpu
from jax.experimental.pallas import tpu_sc as plsc

N = 10000
D = 128
DH = D // 2
NPAD = 10240
NC, NS, L = 2, 16, 16
NW = NC * NS
CHUNK = 128


def _ceil_to(x, m):
    return (x + m - 1) // m * m


def _deg_body(nchunk, dst_hbm, ew_hbm, out_hbm, dst_v, ew_v, zero_v, deg_sp):
    cid = lax.axis_index("c")
    sid = lax.axis_index("s")
    wid = cid * NS + sid
    pltpu.sync_copy(dst_hbm.at[wid], dst_v)
    pltpu.sync_copy(ew_hbm.at[wid], ew_v)

    nz = NPAD // NS

    def _z(i, _):
        zero_v[pl.ds(i * L, L)] = jnp.zeros((L,), jnp.float32)
        return 0

    lax.fori_loop(0, nz // L, _z, 0)
    pltpu.sync_copy(zero_v, deg_sp.at[pl.ds(sid * nz, nz)])
    plsc.subcore_barrier()

    def _chunk(k, _):
        pltpu.sync_copy(ew_v.at[k], deg_sp.at[dst_v.at[k]], add=True)
        return 0

    lax.fori_loop(0, nchunk, _chunk, 0)
    plsc.subcore_barrier()

    @pl.when(sid == 0)
    def _():
        pltpu.sync_copy(deg_sp, out_hbm.at[cid])


def _prop_body(nchunk, src_hbm, dst_hbm, ew_hbm, dis_hbm, h_hbm, b_hbm,
               out_hbm, src_v, dst_v, ew_v, dis_v, idx_v, norm_v, rows_v,
               brow_v, b_v, acc_sp, sem):
    cid = lax.axis_index("c")
    sid = lax.axis_index("s")
    pltpu.sync_copy(src_hbm.at[sid], src_v)
    pltpu.sync_copy(dst_hbm.at[sid], dst_v)
    pltpu.sync_copy(ew_hbm.at[sid], ew_v)
    pltpu.sync_copy(dis_hbm, dis_v)
    pltpu.sync_copy(b_hbm.at[cid], b_v)

    bvecs = [b_v[pl.ds(q * L, L)] for q in range(DH // L)]

    def _br(i, _):
        for q in range(DH // L):
            brow_v[i, pl.ds(q * L, L)] = bvecs[q]
        return 0

    lax.fori_loop(0, CHUNK, _br, 0)
    for dblk in range(NPAD // NS // CHUNK):
        pltpu.sync_copy(brow_v,
                        acc_sp.at[pl.ds(sid * (NPAD // NS) + dblk * CHUNK, CHUNK)])
    plsc.subcore_barrier()

    table_off = cid * N

    def _chunk(k, _):
        for g in range(CHUNK // L):
            idx_v[pl.ds(g * L, L)] = src_v[k, pl.ds(g * L, L)] + table_off
        cp = pltpu.async_copy(h_hbm.at[idx_v], rows_v, sem)
        for g in range(CHUNK // L):
            s16 = src_v[k, pl.ds(g * L, L)]
            d16 = dst_v[k, pl.ds(g * L, L)]
            e16 = ew_v[k, pl.ds(g * L, L)]
            n16 = plsc.load_gather(dis_v, [s16]) * e16 * plsc.load_gather(dis_v, [d16])
            norm_v[pl.ds(g * L, L)] = n16
        cp.wait()

        for g in range(CHUNK // L):
            n16 = norm_v[pl.ds(g * L, L)]
            for lane in range(L):
                s = n16[lane]
                jj = g * L + lane
                for q in range(DH // L):
                    rows_v[jj, pl.ds(q * L, L)] = rows_v[jj, pl.ds(q * L, L)] * s
        pltpu.sync_copy(rows_v, acc_sp.at[dst_v.at[k]], add=True)
        return 0

    lax.fori_loop(0, nchunk, _chunk, 0)
    plsc.subcore_barrier()
    rows_per_tile = NPAD // NS
    pltpu.sync_copy(acc_sp.at[pl.ds(sid * rows_per_tile, rows_per_tile)],
                    out_hbm.at[cid, pl.ds(sid * rows_per_tile, rows_per_tile)])


def _mm_body(parts_ref, x_ref, w_ref, dis_ref, h_ref):
    deg = jnp.sum(parts_ref[...], axis=0, keepdims=True)
    dis = jnp.where(deg > 0.0, lax.rsqrt(jnp.maximum(deg, 1e-12)), 0.0)
    dis_ref[...] = dis
    h = jnp.dot(x_ref[...], w_ref[...], preferred_element_type=jnp.float32)
    h_ref[0] = h[:, :DH]
    h_ref[1] = h[:, DH:]


def kernel(x, edge_index, edge_weight, W, b):
    E = edge_index.shape[1]
    src = edge_index[0].astype(jnp.int32)
    dst = edge_index[1].astype(jnp.int32)
    loop_idx = jnp.arange(N, dtype=jnp.int32)
    e_tot = E + N
    e_pad = _ceil_to(e_tot, NS * CHUNK)
    npad_e = e_pad - e_tot
    src_all = jnp.concatenate([src, loop_idx,
                               jnp.zeros((npad_e,), jnp.int32)])
    dst_all = jnp.concatenate([dst, loop_idx,
                               jnp.zeros((npad_e,), jnp.int32)])
    ew_all = jnp.concatenate([edge_weight.astype(jnp.float32),
                              jnp.ones((N,), jnp.float32),
                              jnp.zeros((npad_e,), jnp.float32)])
    nchunk32 = e_pad // (NW * CHUNK)
    nchunk16 = e_pad // (NS * CHUNK)
    src_r16 = src_all.reshape(NS, nchunk16, CHUNK)
    dst_r16 = dst_all.reshape(NS, nchunk16, CHUNK)
    ew_r16 = ew_all.reshape(NS, nchunk16, CHUNK)
    dst_r32 = dst_all.reshape(NW, nchunk32, CHUNK)
    ew_r32 = ew_all.reshape(NW, nchunk32, CHUNK)

    mesh = plsc.VectorSubcoreMesh(core_axis_name="c", subcore_axis_name="s")

    deg_parts = pl.kernel(
        functools.partial(_deg_body, nchunk32),
        out_type=jax.ShapeDtypeStruct((NC, NPAD), jnp.float32),
        mesh=mesh,
        scratch_types=[
            pltpu.VMEM((nchunk32, CHUNK), jnp.int32),
            pltpu.VMEM((nchunk32, CHUNK), jnp.float32),
            pltpu.VMEM((NPAD // NS,), jnp.float32),
            pltpu.VMEM_SHARED((NPAD,), jnp.float32),
        ],
    )(dst_r32, ew_r32)

    dis2d, h_half = pl.pallas_call(
        _mm_body,
        out_shape=[
            jax.ShapeDtypeStruct((1, NPAD), jnp.float32),
            jax.ShapeDtypeStruct((NC, N, DH), jnp.float32),
        ],
    )(deg_parts, x, W)
    dis = dis2d.reshape(NPAD)
    h_tab = h_half.reshape(NC * N, DH)
    b_r = b.astype(jnp.float32).reshape(NC, DH)

    acc_parts = pl.kernel(
        functools.partial(_prop_body, nchunk16),
        out_type=jax.ShapeDtypeStruct((NC, NPAD, DH), jnp.float32),
        mesh=mesh,
        compiler_params=pltpu.CompilerParams(needs_layout_passes=False,
                                             use_tc_tiling_on_sc=False),
        scratch_types=[
            pltpu.VMEM((nchunk16, CHUNK), jnp.int32),
            pltpu.VMEM((nchunk16, CHUNK), jnp.int32),
            pltpu.VMEM((nchunk16, CHUNK), jnp.float32),
            pltpu.VMEM((NPAD,), jnp.float32),
            pltpu.VMEM((CHUNK,), jnp.int32),
            pltpu.VMEM((CHUNK,), jnp.float32),
            pltpu.VMEM((CHUNK, DH), jnp.float32),
            pltpu.VMEM((CHUNK, DH), jnp.float32),
            pltpu.VMEM((DH,), jnp.float32),
            pltpu.VMEM_SHARED((NPAD, DH), jnp.float32),
            pltpu.SemaphoreType.DMA,
        ],
    )(src_r16, dst_r16, ew_r16, dis, h_tab, b_r)

    out = jnp.concatenate([acc_parts[0, :N], acc_parts[1, :N]], axis=1)
    return out
